# Initial kernel scaffold; baseline (speedup 1.0000x reference)
import jax
import jax.numpy as jnp
from jax.experimental import pallas as pl


def kernel(h_neigh, h_self, edge_features, W_self, W_neigh, W_edge, b_edge, edge_index):
    raise NotImplementedError("write your pallas kernel here")


# speedup vs baseline: 77.0821x; 77.0821x over previous
import functools

import jax
import jax.numpy as jnp
from jax import lax
from jax.experimental import pallas as pl
from jax.experimental.pallas import tpu as pltpu
from jax.experimental.pallas import tpu_sc as plsc

N_NODES = 10000
N_EDGES = 160000
D = 16

NC = 2
NS = 16
NW = NC * NS
C = 128
U = 8
EW_BASE = (N_EDGES // NW) // C * C
N_EXTRA = (N_EDGES - EW_BASE * NW) // C
N_PAD = 10240
ROWS_PT = N_PAD // NS


def _ew_body(ef_ref, w_ref, b_ref, out_ref):
    out_ref[...] = (
        jnp.dot(ef_ref[...], w_ref[...], preferred_element_type=jnp.float32)
        + b_ref[...]
    )


def _ew_call(ef_packed, w_big, b_big):
    m = ef_packed.shape[0]
    blk = 2000
    grid = m // blk
    return pl.pallas_call(
        _ew_body,
        grid=(grid,),
        in_specs=[
            pl.BlockSpec((blk, 128), lambda i: (i, 0)),
            pl.BlockSpec((128, 128), lambda i: (0, 0)),
            pl.BlockSpec((1, 128), lambda i: (0, 0)),
        ],
        out_specs=pl.BlockSpec((blk, 128), lambda i: (i, 0)),
        out_shape=jax.ShapeDtypeStruct((m, 128), jnp.float32),
    )(ef_packed, w_big, b_big)


def _sc_body(h_hbm, ew_hbm, src_hbm, dst_hbm, zeros_hbm, out_hbm,
             sidx, didx, gath, ewv, msg, acc, sem):
    cid = lax.axis_index("c")
    sid = lax.axis_index("s")
    wid = sid * NC + cid

    pltpu.sync_copy(zeros_hbm, acc.at[pl.ds(sid * ROWS_PT, ROWS_PT)])

    one_hot = jnp.where(
        lax.broadcasted_iota(jnp.int32, (D,), 0) == 0, 1.0, 0.0
    ).astype(jnp.float32)

    zeros16 = jnp.zeros((D,), jnp.float32)

    def _init_row(i, _):
        msg[i, pl.ds(D, D)] = one_hot
        for k in range(2 * D, 128, D):
            msg[i, pl.ds(k, D)] = zeros16
        return 0

    lax.fori_loop(0, C, _init_row, 0)
    plsc.subcore_barrier()

    def _do_chunk(base):
        pltpu.sync_copy(src_hbm.at[pl.ds(base, C)], sidx)
        cp = pltpu.async_copy(h_hbm.at[sidx], gath, sem)
        pbase = pl.multiple_of(base // 8, 8)
        pltpu.sync_copy(ew_hbm.at[pl.ds(pbase, C // 8)], ewv)
        pltpu.sync_copy(dst_hbm.at[pl.ds(base, C)], didx)
        cp.wait()

        def _rows(i, _):
            for u in range(U):
                r = i * U + u
                msg[r, pl.ds(0, D)] = gath[r, pl.ds(0, D)] * ewv[i, pl.ds(u * D, D)]
            return 0

        lax.fori_loop(0, C // U, _rows, 0)
        pltpu.sync_copy(msg, acc.at[didx], add=True)

    def _chunk_step(i, _):
        _do_chunk(wid * EW_BASE + i * C)
        return 0

    lax.fori_loop(0, EW_BASE // C, _chunk_step, 0)

    @pl.when(wid < N_EXTRA)
    def _():
        _do_chunk(NW * EW_BASE + wid * C)

    plsc.subcore_barrier()
    pltpu.sync_copy(
        acc.at[pl.ds(sid * ROWS_PT, ROWS_PT)],
        out_hbm.at[pl.ds(cid * N_PAD + sid * ROWS_PT, ROWS_PT)],
    )


_sc_call = functools.partial(
    pl.kernel,
    out_type=jax.ShapeDtypeStruct((NC * N_PAD, 128), jnp.float32),
    mesh=plsc.VectorSubcoreMesh(core_axis_name="c", subcore_axis_name="s",
                                num_cores=NC, num_subcores=NS),
    scratch_types=[
        pltpu.VMEM((C,), jnp.int32),
        pltpu.VMEM((C,), jnp.int32),
        pltpu.VMEM((C, 128), jnp.float32),
        pltpu.VMEM((C // 8, 128), jnp.float32),
        pltpu.VMEM((C, 128), jnp.float32),
        pltpu.VMEM_SHARED((N_PAD, 128), jnp.float32),
        pltpu.SemaphoreType.DMA,
    ],
)(_sc_body)


def _fin_body(p0_ref, p1_ref, h_ref, ws_ref, wn_ref, out_ref):
    s = p0_ref[...] + p1_ref[...]
    sums = s[:, :D]
    cnt = s[:, D:D + 1]
    agg = sums / jnp.maximum(cnt, 1.0)
    z = (
        jnp.dot(h_ref[...], ws_ref[...], preferred_element_type=jnp.float32)
        + jnp.dot(agg, wn_ref[...], preferred_element_type=jnp.float32)
    )
    out_ref[...] = jnp.maximum(z, 0.0)


def _fin_call(p0, p1, h_self, wsT, wnT):
    blk = 2000
    grid = N_NODES // blk
    return pl.pallas_call(
        _fin_body,
        grid=(grid,),
        in_specs=[
            pl.BlockSpec((blk, 128), lambda i: (i, 0)),
            pl.BlockSpec((blk, 128), lambda i: (i, 0)),
            pl.BlockSpec((blk, D), lambda i: (i, 0)),
            pl.BlockSpec((D, D), lambda i: (0, 0)),
            pl.BlockSpec((D, D), lambda i: (0, 0)),
        ],
        out_specs=pl.BlockSpec((blk, D), lambda i: (i, 0)),
        out_shape=jax.ShapeDtypeStruct((N_NODES, D), jnp.float32),
    )(p0, p1, h_self, wsT, wnT)


def kernel(h_neigh, h_self, edge_features, W_self, W_neigh, W_edge, b_edge,
           edge_index):
    src = edge_index[0].astype(jnp.int32)
    dst = edge_index[1].astype(jnp.int32)

    w_red = W_edge.reshape(D, D, D).sum(axis=0)
    w_big = jnp.kron(jnp.eye(8, dtype=jnp.float32), w_red.T)
    b_big = jnp.tile(b_edge.reshape(D, D).sum(axis=0), 8).reshape(1, 128)

    ef_packed = edge_features.reshape(N_EDGES // 8, 128)
    ew_packed = _ew_call(ef_packed, w_big, b_big)

    h_pad = jnp.pad(h_neigh, ((0, 0), (0, 128 - D)))
    zeros = jnp.zeros((ROWS_PT, 128), jnp.float32)
    part = _sc_call(h_pad, ew_packed, src, dst, zeros)
    p0 = part[:N_NODES]
    p1 = part[N_PAD:N_PAD + N_NODES]

    return _fin_call(p0, p1, h_self, W_self.T, W_neigh.T)


# speedup vs baseline: 85.6793x; 1.1115x over previous
import functools

import jax
import jax.numpy as jnp
from jax import lax
from jax.experimental import pallas as pl
from jax.experimental.pallas import tpu as pltpu
from jax.experimental.pallas import tpu_sc as plsc

N_NODES = 10000
N_EDGES = 160000
D = 16

NC = 2
NS = 16
NW = NC * NS
C = 128
U = 8
EW_BASE = (N_EDGES // NW) // C * C
N_EXTRA = (N_EDGES - EW_BASE * NW) // C
N_PAD = 10240
ROWS_PT = N_PAD // NS


def _ew_body(ef_ref, w_ref, b_ref, out_ref):
    out_ref[...] = (
        jnp.dot(ef_ref[...], w_ref[...], preferred_element_type=jnp.float32)
        + b_ref[...]
    )


def _ew_call(ef_packed, w_big, b_big):
    m = ef_packed.shape[0]
    blk = 2000
    grid = m // blk
    return pl.pallas_call(
        _ew_body,
        grid=(grid,),
        in_specs=[
            pl.BlockSpec((blk, 128), lambda i: (i, 0)),
            pl.BlockSpec((128, 128), lambda i: (0, 0)),
            pl.BlockSpec((1, 128), lambda i: (0, 0)),
        ],
        out_specs=pl.BlockSpec((blk, 128), lambda i: (i, 0)),
        out_shape=jax.ShapeDtypeStruct((m, 128), jnp.float32),
    )(ef_packed, w_big, b_big)


def _sc_body(h_hbm, ew_hbm, src_hbm, dst_hbm, zeros_hbm, out_hbm,
             sidx, didx, gath, ewv, msg, acc, sem):
    cid = lax.axis_index("c")
    sid = lax.axis_index("s")
    wid = sid * NC + cid

    pltpu.sync_copy(zeros_hbm, acc.at[pl.ds(sid * ROWS_PT, ROWS_PT)])

    one_hot = jnp.where(
        lax.broadcasted_iota(jnp.int32, (D,), 0) == 0, 1.0, 0.0
    ).astype(jnp.float32)

    def _init_row(i, _):
        msg[i, pl.ds(D, D)] = one_hot
        return 0

    lax.fori_loop(0, C, _init_row, 0)
    plsc.subcore_barrier()

    def _do_chunk(base):
        pltpu.sync_copy(src_hbm.at[pl.ds(base, C)], sidx)
        cp = pltpu.async_copy(h_hbm.at[sidx], gath, sem)
        pbase = pl.multiple_of(base // 8, 8)
        pltpu.sync_copy(ew_hbm.at[pl.ds(pbase, C // 8)], ewv)
        pltpu.sync_copy(dst_hbm.at[pl.ds(base, C)], didx)
        cp.wait()

        def _rows(i, _):
            for u in range(U):
                r = i * U + u
                msg[r, pl.ds(0, D)] = gath[r, :] * ewv[i, pl.ds(u * D, D)]
            return 0

        lax.fori_loop(0, C // U, _rows, 0)
        pltpu.sync_copy(msg, acc.at[didx], add=True)

    def _chunk_step(i, _):
        _do_chunk(wid * EW_BASE + i * C)
        return 0

    lax.fori_loop(0, EW_BASE // C, _chunk_step, 0)

    @pl.when(wid < N_EXTRA)
    def _():
        _do_chunk(NW * EW_BASE + wid * C)

    plsc.subcore_barrier()
    pltpu.sync_copy(
        acc.at[pl.ds(sid * ROWS_PT, ROWS_PT)],
        out_hbm.at[pl.ds(cid * N_PAD + sid * ROWS_PT, ROWS_PT)],
    )


_sc_call = functools.partial(
    pl.kernel,
    out_type=jax.ShapeDtypeStruct((NC * N_PAD, 2 * D), jnp.float32),
    mesh=plsc.VectorSubcoreMesh(core_axis_name="c", subcore_axis_name="s",
                                num_cores=NC, num_subcores=NS),
    scratch_types=[
        pltpu.VMEM((C,), jnp.int32),
        pltpu.VMEM((C,), jnp.int32),
        pltpu.VMEM((C, D), jnp.float32),
        pltpu.VMEM((C // 8, 128), jnp.float32),
        pltpu.VMEM((C, 2 * D), jnp.float32),
        pltpu.VMEM_SHARED((N_PAD, 2 * D), jnp.float32),
        pltpu.SemaphoreType.DMA,
    ],
    compiler_params=pltpu.CompilerParams(use_tc_tiling_on_sc=False),
)(_sc_body)


def _fin_body(p0_ref, p1_ref, h_ref, ws_ref, wn_ref, out_ref):
    s = p0_ref[...] + p1_ref[...]
    sums = s[:, :D]
    cnt = s[:, D:D + 1]
    agg = sums / jnp.maximum(cnt, 1.0)
    z = (
        jnp.dot(h_ref[...], ws_ref[...], preferred_element_type=jnp.float32)
        + jnp.dot(agg, wn_ref[...], preferred_element_type=jnp.float32)
    )
    out_ref[...] = jnp.maximum(z, 0.0)


def _fin_call(p0, p1, h_self, wsT, wnT):
    blk = 2000
    grid = N_NODES // blk
    return pl.pallas_call(
        _fin_body,
        grid=(grid,),
        in_specs=[
            pl.BlockSpec((blk, 2 * D), lambda i: (i, 0)),
            pl.BlockSpec((blk, 2 * D), lambda i: (i, 0)),
            pl.BlockSpec((blk, D), lambda i: (i, 0)),
            pl.BlockSpec((D, D), lambda i: (0, 0)),
            pl.BlockSpec((D, D), lambda i: (0, 0)),
        ],
        out_specs=pl.BlockSpec((blk, D), lambda i: (i, 0)),
        out_shape=jax.ShapeDtypeStruct((N_NODES, D), jnp.float32),
    )(p0, p1, h_self, wsT, wnT)


def kernel(h_neigh, h_self, edge_features, W_self, W_neigh, W_edge, b_edge,
           edge_index):
    src = edge_index[0].astype(jnp.int32)
    dst = edge_index[1].astype(jnp.int32)

    w_red = W_edge.reshape(D, D, D).sum(axis=0)
    w_big = jnp.kron(jnp.eye(8, dtype=jnp.float32), w_red.T)
    b_big = jnp.tile(b_edge.reshape(D, D).sum(axis=0), 8).reshape(1, 128)

    ef_packed = edge_features.reshape(N_EDGES // 8, 128)
    ew_packed = _ew_call(ef_packed, w_big, b_big)

    zeros = jnp.zeros((ROWS_PT, 2 * D), jnp.float32)
    part = _sc_call(h_neigh, ew_packed, src, dst, zeros)
    p0 = part[:N_NODES]
    p1 = part[N_PAD:N_PAD + N_NODES]

    return _fin_call(p0, p1, h_self, W_self.T, W_neigh.T)


# speedup vs baseline: 101.7929x; 1.1881x over previous
import functools

import jax
import jax.numpy as jnp
from jax import lax
from jax.experimental import pallas as pl
from jax.experimental.pallas import tpu as pltpu
from jax.experimental.pallas import tpu_sc as plsc

N_NODES = 10000
N_EDGES = 160000
D = 16

NC = 2
NS = 16
NW = NC * NS
C = 512
U = 8
EW_PER = 5120
E_PAD = EW_PER * NW
NCH = EW_PER // C
N_PAD = 10240
ROWS_PT = N_PAD // NS


def _ew_body(ef_ref, w_ref, b_ref, out_ref):
    out_ref[...] = (
        jnp.dot(ef_ref[...], w_ref[...], preferred_element_type=jnp.float32)
        + b_ref[...]
    )


def _ew_call(ef_packed, w_big, b_big):
    m = ef_packed.shape[0]
    blk = 2048
    grid = m // blk
    return pl.pallas_call(
        _ew_body,
        grid=(grid,),
        in_specs=[
            pl.BlockSpec((blk, 128), lambda i: (i, 0)),
            pl.BlockSpec((128, 128), lambda i: (0, 0)),
            pl.BlockSpec((1, 128), lambda i: (0, 0)),
        ],
        out_specs=pl.BlockSpec((blk, 128), lambda i: (i, 0)),
        out_shape=jax.ShapeDtypeStruct((m, 128), jnp.float32),
    )(ef_packed, w_big, b_big)


def _sc_body(h_hbm, ew_hbm, src_hbm, dst_hbm, zeros_hbm, out_hbm,
             src_all, dst_all, gath0, gath1, ewv0, ewv1, msg0, msg1, acc,
             sg0, sg1, se0, se1):
    cid = lax.axis_index("c")
    sid = lax.axis_index("s")
    wid = sid * NC + cid
    gath = (gath0, gath1)
    ewv = (ewv0, ewv1)
    msg = (msg0, msg1)
    sg = (sg0, sg1)
    se = (se0, se1)

    pltpu.sync_copy(zeros_hbm, acc.at[pl.ds(sid * ROWS_PT, ROWS_PT)])

    ebase = pl.multiple_of(wid * EW_PER, 8)
    pltpu.sync_copy(src_hbm.at[pl.ds(ebase, EW_PER)], src_all)
    pltpu.sync_copy(dst_hbm.at[pl.ds(ebase, EW_PER)], dst_all)

    one_hot = jnp.where(
        lax.broadcasted_iota(jnp.int32, (D,), 0) == 0, 1.0, 0.0
    ).astype(jnp.float32)

    def _init_row(i, _):
        msg0[i, pl.ds(D, D)] = one_hot
        msg1[i, pl.ds(D, D)] = one_hot
        return 0

    lax.fori_loop(0, C, _init_row, 0)
    plsc.subcore_barrier()

    pwbase = pl.multiple_of(wid * (EW_PER // 8), 8)

    def _issue(i, s):
        pltpu.async_copy(h_hbm.at[src_all.at[pl.ds(i * C, C)]], gath[s], sg[s])
        poff = pl.multiple_of(pwbase + i * (C // 8), 8)
        pltpu.async_copy(ew_hbm.at[pl.ds(poff, C // 8)], ewv[s], se[s])

    def _wait(s):
        pltpu.make_async_copy(h_hbm, gath[s], sg[s]).wait()
        pltpu.make_async_copy(ew_hbm, ewv[s], se[s]).wait()

    def _compute(s):
        g, e, m = gath[s], ewv[s], msg[s]

        def _rows(i, _):
            for u in range(U):
                r = i * U + u
                m[r, pl.ds(0, D)] = g[r, :] * e[i, pl.ds(u * D, D)]
            return 0

        lax.fori_loop(0, C // U, _rows, 0)

    def _scatter(i, s):
        pltpu.sync_copy(msg[s], acc.at[dst_all.at[pl.ds(i * C, C)]], add=True)

    _issue(0, 0)

    def _pair(j, _):
        i0 = 2 * j
        _issue(i0 + 1, 1)
        _wait(0)
        _compute(0)

        @pl.when(j < NCH // 2 - 1)
        def _():
            _issue(i0 + 2, 0)

        _scatter(i0, 0)
        _wait(1)
        _compute(1)

        @pl.when(j < NCH // 2 - 1)
        def _():
            _issue(i0 + 3, 1)

        _scatter(i0 + 1, 1)
        return 0

    lax.fori_loop(0, NCH // 2, _pair, 0)

    plsc.subcore_barrier()
    pltpu.sync_copy(
        acc.at[pl.ds(sid * ROWS_PT, ROWS_PT)],
        out_hbm.at[pl.ds(cid * N_PAD + sid * ROWS_PT, ROWS_PT)],
    )


_sc_call = functools.partial(
    pl.kernel,
    out_type=jax.ShapeDtypeStruct((NC * N_PAD, 2 * D), jnp.float32),
    mesh=plsc.VectorSubcoreMesh(core_axis_name="c", subcore_axis_name="s",
                                num_cores=NC, num_subcores=NS),
    scratch_types=[
        pltpu.VMEM((EW_PER,), jnp.int32),
        pltpu.VMEM((EW_PER,), jnp.int32),
        pltpu.VMEM((C, D), jnp.float32),
        pltpu.VMEM((C, D), jnp.float32),
        pltpu.VMEM((C // 8, 128), jnp.float32),
        pltpu.VMEM((C // 8, 128), jnp.float32),
        pltpu.VMEM((C, 2 * D), jnp.float32),
        pltpu.VMEM((C, 2 * D), jnp.float32),
        pltpu.VMEM_SHARED((N_PAD, 2 * D), jnp.float32),
        pltpu.SemaphoreType.DMA,
        pltpu.SemaphoreType.DMA,
        pltpu.SemaphoreType.DMA,
        pltpu.SemaphoreType.DMA,
    ],
    compiler_params=pltpu.CompilerParams(use_tc_tiling_on_sc=False),
)(_sc_body)


def _fin_body(p0_ref, p1_ref, h_ref, ws_ref, wn_ref, out_ref):
    s = p0_ref[...] + p1_ref[...]
    sums = s[:, :D]
    cnt = s[:, D:D + 1]
    agg = sums / jnp.maximum(cnt, 1.0)
    z = (
        jnp.dot(h_ref[...], ws_ref[...], preferred_element_type=jnp.float32)
        + jnp.dot(agg, wn_ref[...], preferred_element_type=jnp.float32)
    )
    out_ref[...] = jnp.maximum(z, 0.0)


def _fin_call(p0, p1, h_self, wsT, wnT):
    blk = 2000
    grid = N_NODES // blk
    return pl.pallas_call(
        _fin_body,
        grid=(grid,),
        in_specs=[
            pl.BlockSpec((blk, 2 * D), lambda i: (i, 0)),
            pl.BlockSpec((blk, 2 * D), lambda i: (i, 0)),
            pl.BlockSpec((blk, D), lambda i: (i, 0)),
            pl.BlockSpec((D, D), lambda i: (0, 0)),
            pl.BlockSpec((D, D), lambda i: (0, 0)),
        ],
        out_specs=pl.BlockSpec((blk, D), lambda i: (i, 0)),
        out_shape=jax.ShapeDtypeStruct((N_NODES, D), jnp.float32),
    )(p0, p1, h_self, wsT, wnT)


def kernel(h_neigh, h_self, edge_features, W_self, W_neigh, W_edge, b_edge,
           edge_index):
    src = edge_index[0].astype(jnp.int32)
    dst = edge_index[1].astype(jnp.int32)
    src = jnp.pad(src, (0, E_PAD - N_EDGES))
    dst = jnp.pad(dst, (0, E_PAD - N_EDGES), constant_values=N_PAD - 1)

    w_red = W_edge.reshape(D, D, D).sum(axis=0)
    w_big = jnp.kron(jnp.eye(8, dtype=jnp.float32), w_red.T)
    b_big = jnp.tile(b_edge.reshape(D, D).sum(axis=0), 8).reshape(1, 128)

    ef_packed = edge_features.reshape(N_EDGES // 8, 128)
    ef_packed = jnp.pad(ef_packed, ((0, (E_PAD - N_EDGES) // 8), (0, 0)))
    ew_packed = _ew_call(ef_packed, w_big, b_big)

    zeros = jnp.zeros((ROWS_PT, 2 * D), jnp.float32)
    part = _sc_call(h_neigh, ew_packed, src, dst, zeros)
    p0 = part[:N_NODES]
    p1 = part[N_PAD:N_PAD + N_NODES]

    return _fin_call(p0, p1, h_self, W_self.T, W_neigh.T)
